# Initial kernel scaffold; baseline (speedup 1.0000x reference)
import jax
import jax.numpy as jnp
from jax.experimental import pallas as pl


def kernel(x):
    raise NotImplementedError("write your pallas kernel here")


# speedup vs baseline: 1.0800x; 1.0800x over previous
import jax
import jax.numpy as jnp
from jax.experimental import pallas as pl
from jax.experimental.pallas import tpu as pltpu

_G = 8


def _copy_kernel(order_ref, x_ref, o_ref):
    o_ref[...] = x_ref[...]


def kernel(x):
    N, C, H, W = x.shape
    g = _G
    perm = jax.random.permutation(jax.random.key(42), g - 1)
    order = jnp.concatenate(
        [perm, jnp.array([g - 1], dtype=perm.dtype)], axis=0
    ).astype(jnp.int32)
    cg = C // g
    hw = H * W
    xr = x.reshape(N, g, cg, hw)
    blk = 12544
    nj = hw // blk
    grid_spec = pltpu.PrefetchScalarGridSpec(
        num_scalar_prefetch=1,
        grid=(N, g, nj),
        in_specs=[
            pl.BlockSpec((1, 1, cg, blk), lambda n, i, j, order_ref: (n, order_ref[i], 0, j))
        ],
        out_specs=pl.BlockSpec((1, 1, cg, blk), lambda n, i, j, order_ref: (n, i, 0, j)),
    )
    out = pl.pallas_call(
        _copy_kernel,
        grid_spec=grid_spec,
        out_shape=jax.ShapeDtypeStruct((N, g, cg, hw), x.dtype),
    )(order, xr)
    return out.reshape(N, C, H, W)
